# Initial kernel scaffold; baseline (speedup 1.0000x reference)
import jax
import jax.numpy as jnp
from jax.experimental import pallas as pl


def kernel(x, segment_ids, pos, Wq, bq, Wk, bk, Wv, bv, Wo, bo):
    raise NotImplementedError("write your pallas kernel here")


# speedup vs baseline: 11.8617x; 11.8617x over previous
import jax
import jax.numpy as jnp
from jax.experimental import pallas as pl
from jax.experimental.pallas import tpu as pltpu

_EMBED = 1024
_HEADS = 16
_HD = _EMBED // _HEADS
_SEQ = 2048
_BLK = 256
_NEG = -1e30


def _dot_t(a, b):
    return jax.lax.dot_general(
        a, b, (((1,), (1,)), ((), ())),
        preferred_element_type=jnp.float32,
        precision=jax.lax.Precision.DEFAULT)


def _dot(a, b):
    return jax.lax.dot_general(
        a, b, (((1,), (0,)), ((), ())),
        preferred_element_type=jnp.float32,
        precision=jax.lax.Precision.DEFAULT)


def _body(pos_ref, ids_ref, x_ref, wq_ref, wk_ref, wv_ref, wo_ref,
          bq_ref, bk_ref, bv_ref, bo_ref, out_ref, k_buf, v_buf):
    pos = pos_ref[0, 0]
    ids = ids_ref[...]
    ri = jax.lax.broadcasted_iota(jnp.int32, ids.shape, 0)
    ci = jax.lax.broadcasted_iota(jnp.int32, ids.shape, 1)
    flat = ri * 128 + ci
    seg = jnp.sum(jnp.where(flat == pos, ids, 0))
    lo = jnp.sum((ids < seg).astype(jnp.int32))
    n = jnp.sum((ids == seg).astype(jnp.int32))
    hi = lo + n
    lo_a = (lo // _BLK) * _BLK
    nblk = (hi + _BLK - 1) // _BLK - lo // _BLK
    nf = n.astype(jnp.float32)

    bq = bq_ref[...]
    bk = bk_ref[...]
    bv = bv_ref[...]
    bo = bo_ref[...]

    xsum = jnp.sum(x_ref[...], axis=0, keepdims=True)
    vsum_all = _dot_t(xsum, wv_ref[...]) + float(_SEQ) * bv

    def phase_a(t, vs):
        base = pl.multiple_of(lo_a + t * _BLK, _BLK)
        xb = x_ref[pl.ds(base, _BLK), :]
        kb = _dot_t(xb, wk_ref[...]) + bk
        vb = _dot_t(xb, wv_ref[...]) + bv
        g = base + jax.lax.broadcasted_iota(jnp.int32, (_BLK, 1), 0)
        ins = (g >= lo) & (g < hi)
        kb = jnp.where(ins, kb, 0.0)
        vb = jnp.where(ins, vb, 0.0)
        k_buf[pl.ds(t * _BLK, _BLK), :] = kb
        v_buf[pl.ds(t * _BLK, _BLK), :] = vb
        return vs + jnp.sum(vb, axis=0, keepdims=True)

    vsum_seg = jax.lax.fori_loop(
        0, nblk, phase_a, jnp.zeros((1, _EMBED), jnp.float32))
    vs_out = vsum_all - vsum_seg
    n_out = float(_SEQ) - nf

    def phase_b(t, acc):
        base = pl.multiple_of(lo_a + t * _BLK, _BLK)
        xb = x_ref[pl.ds(base, _BLK), :]
        qb = _dot_t(xb, wq_ref[...]) + bq
        g = base + jax.lax.broadcasted_iota(jnp.int32, (_BLK, 1), 0)
        q_ins = (g >= lo) & (g < hi)
        outs = []
        for h in range(_HEADS):
            sl = slice(h * _HD, (h + 1) * _HD)
            qh = qb[:, sl]

            def inner(u, carry, qh=qh, sl=sl):
                m, l, a = carry
                kh = k_buf[pl.ds(u * _BLK, _BLK), sl]
                vh = v_buf[pl.ds(u * _BLK, _BLK), sl]
                s = _dot_t(qh, kh)
                cg = (lo_a + u * _BLK
                      + jax.lax.broadcasted_iota(jnp.int32, (1, _BLK), 1))
                cv = (cg >= lo) & (cg < hi)
                sm = jnp.where(cv, s, _NEG)
                m_new = jnp.maximum(m, jnp.max(sm, axis=1, keepdims=True))
                p = jnp.where(cv, jnp.exp(sm - m_new), 0.0)
                corr = jnp.exp(m - m_new)
                a = a * corr + _dot(p, vh)
                l = l * corr + jnp.sum(p, axis=1, keepdims=True)
                return m_new, l, a

            m0 = jnp.full((_BLK, 1), _NEG, jnp.float32)
            l0 = jnp.zeros((_BLK, 1), jnp.float32)
            a0 = jnp.zeros((_BLK, _HD), jnp.float32)
            m, l, a = jax.lax.fori_loop(0, nblk, inner, (m0, l0, a0))
            m_f = jnp.maximum(m, 0.0)
            c1 = jnp.exp(m - m_f)
            c0 = jnp.exp(-m_f)
            num = a * c1 + c0 * vs_out[:, sl]
            den = l * c1 + c0 * n_out
            ctx = jnp.where(q_ins, num / den, 0.0)
            outs.append(jnp.sum(ctx, axis=0, keepdims=True))
        return acc + jnp.concatenate(outs, axis=1)

    acc = jax.lax.fori_loop(
        0, nblk, phase_b, jnp.zeros((1, _EMBED), jnp.float32))
    out_ref[...] = _dot_t(acc / nf, wo_ref[...]) + bo


def _call(pos_arr, ids2, x, wq, wk, wv, wo, bq2, bk2, bv2, bo2,
          interpret=False):
    return pl.pallas_call(
        _body,
        out_shape=jax.ShapeDtypeStruct((1, _EMBED), jnp.float32),
        in_specs=[
            pl.BlockSpec(memory_space=pltpu.SMEM),
            pl.BlockSpec(),
            pl.BlockSpec(),
            pl.BlockSpec(), pl.BlockSpec(), pl.BlockSpec(), pl.BlockSpec(),
            pl.BlockSpec(), pl.BlockSpec(), pl.BlockSpec(), pl.BlockSpec(),
        ],
        scratch_shapes=[
            pltpu.VMEM((_SEQ, _EMBED), jnp.float32),
            pltpu.VMEM((_SEQ, _EMBED), jnp.float32),
        ],
        interpret=interpret,
    )(pos_arr, ids2, x, wq, wk, wv, wo, bq2, bk2, bv2, bo2)


def kernel(x, segment_ids, pos, Wq, bq, Wk, bk, Wv, bv, Wo, bo):
    pos_arr = jnp.asarray(pos, jnp.int32).reshape(1, 1)
    ids2 = jnp.asarray(segment_ids, jnp.int32).reshape(_SEQ // 128, 128)
    out = _call(pos_arr, ids2, x,
                Wq, Wk, Wv, Wo,
                bq.reshape(1, _EMBED), bk.reshape(1, _EMBED),
                bv.reshape(1, _EMBED), bo.reshape(1, _EMBED))
    return out.reshape(_EMBED)


# speedup vs baseline: 15.7136x; 1.3247x over previous
import jax
import jax.numpy as jnp
from jax.experimental import pallas as pl
from jax.experimental.pallas import tpu as pltpu

_EMBED = 1024
_HEADS = 16
_HD = _EMBED // _HEADS
_SEQ = 2048
_BLK = 256
_CAP = 256
_NEG = -1e30


def _dot_t(a, b):
    return jax.lax.dot_general(
        a, b, (((1,), (1,)), ((), ())),
        preferred_element_type=jnp.float32,
        precision=jax.lax.Precision.DEFAULT)


def _dot(a, b):
    return jax.lax.dot_general(
        a, b, (((1,), (0,)), ((), ())),
        preferred_element_type=jnp.float32,
        precision=jax.lax.Precision.DEFAULT)


def _body(pos_ref, ids_ref, x_ref, wq_ref, wk_ref, wv_ref, wo_ref,
          bq_ref, bk_ref, bv_ref, bo_ref, out_ref,
          k_buf, v_buf, q_buf, s_buf, acc_ref):
    pos = pos_ref[0, 0]
    ids = ids_ref[...]
    ri = jax.lax.broadcasted_iota(jnp.int32, ids.shape, 0)
    ci = jax.lax.broadcasted_iota(jnp.int32, ids.shape, 1)
    flat = ri * 128 + ci
    seg = jnp.sum(jnp.where(flat == pos, ids, 0))
    lo = jnp.sum((ids < seg).astype(jnp.int32))
    n = jnp.sum((ids == seg).astype(jnp.int32))
    hi = lo + n
    nf = n.astype(jnp.float32)
    n_out = float(_SEQ) - nf

    bq = bq_ref[...]
    bk = bk_ref[...]
    bv = bv_ref[...]

    xsum = jnp.sum(x_ref[...], axis=0, keepdims=True)
    vsum_all = _dot_t(xsum, wv_ref[...]) + float(_SEQ) * bv

    @pl.when(n <= _CAP - 7)
    def _fast():
        st = (jnp.minimum(lo, _SEQ - _CAP) // 8) * 8
        xw = x_ref[pl.ds(st, _CAP), :]
        q_buf[...] = _dot_t(xw, wq_ref[...]) + bq
        k_buf[pl.ds(0, _CAP), :] = _dot_t(xw, wk_ref[...]) + bk
        v = _dot_t(xw, wv_ref[...]) + bv
        v_buf[pl.ds(0, _CAP), :] = v
        g = st + jax.lax.broadcasted_iota(jnp.int32, (_CAP, 1), 0)
        ins = (g >= lo) & (g < hi)
        vsum_seg = jnp.sum(jnp.where(ins, v, 0.0), axis=0, keepdims=True)
        vs_out = vsum_all - vsum_seg

        for h in range(_HEADS):
            qh = q_buf[:, h * _HD:(h + 1) * _HD]
            kh = k_buf[pl.ds(0, _CAP), h * _HD:(h + 1) * _HD]
            s_buf[:, h * _CAP:(h + 1) * _CAP] = _dot_t(qh, kh)

        lane = jax.lax.broadcasted_iota(jnp.int32, (1, _HEADS * _CAP), 1)
        gcol = st + (lane & (_CAP - 1))
        cv = (gcol >= lo) & (gcol < hi)
        s = s_buf[...]
        sm = jnp.where(cv, s, _NEG)
        m3 = jnp.max(sm.reshape(_CAP, _HEADS, _CAP), axis=2)
        m = jnp.maximum(m3, 0.0)
        mb = jnp.broadcast_to(
            m.reshape(_CAP, _HEADS, 1), (_CAP, _HEADS, _CAP)
        ).reshape(_CAP, _HEADS * _CAP)
        p = jnp.where(cv, jnp.exp(sm - mb), 0.0)
        l = jnp.sum(p.reshape(_CAP, _HEADS, _CAP), axis=2)
        em = jnp.exp(-m)
        den = l + em * n_out
        inv = jnp.where(ins, 1.0 / den, 0.0)
        invb = jnp.broadcast_to(
            inv.reshape(_CAP, _HEADS, 1), (_CAP, _HEADS, _CAP)
        ).reshape(_CAP, _HEADS * _CAP)
        r = jnp.sum(p * invb, axis=0, keepdims=True)
        alpha = jnp.sum(inv * em, axis=0, keepdims=True)
        outs = []
        for h in range(_HEADS):
            rh = r[:, h * _CAP:(h + 1) * _CAP]
            vh = v_buf[pl.ds(0, _CAP), h * _HD:(h + 1) * _HD]
            outs.append(_dot(rh, vh))
        aterm = (vs_out.reshape(1, _HEADS, _HD)
                 * alpha.reshape(1, _HEADS, 1)).reshape(1, _EMBED)
        acc_ref[...] = jnp.concatenate(outs, axis=1) + aterm

    @pl.when(n > _CAP - 7)
    def _general():
        lo_a = (lo // _BLK) * _BLK
        nblk = (hi + _BLK - 1) // _BLK - lo // _BLK

        def phase_a(t, vs):
            base = pl.multiple_of(lo_a + t * _BLK, _BLK)
            xb = x_ref[pl.ds(base, _BLK), :]
            kb = _dot_t(xb, wk_ref[...]) + bk
            vb = _dot_t(xb, wv_ref[...]) + bv
            g = base + jax.lax.broadcasted_iota(jnp.int32, (_BLK, 1), 0)
            ins = (g >= lo) & (g < hi)
            kb = jnp.where(ins, kb, 0.0)
            vb = jnp.where(ins, vb, 0.0)
            k_buf[pl.ds(t * _BLK, _BLK), :] = kb
            v_buf[pl.ds(t * _BLK, _BLK), :] = vb
            return vs + jnp.sum(vb, axis=0, keepdims=True)

        vsum_seg = jax.lax.fori_loop(
            0, nblk, phase_a, jnp.zeros((1, _EMBED), jnp.float32))
        vs_out = vsum_all - vsum_seg

        def phase_b(t, acc):
            base = pl.multiple_of(lo_a + t * _BLK, _BLK)
            xb = x_ref[pl.ds(base, _BLK), :]
            qb = _dot_t(xb, wq_ref[...]) + bq
            g = base + jax.lax.broadcasted_iota(jnp.int32, (_BLK, 1), 0)
            q_ins = (g >= lo) & (g < hi)
            outs = []
            for h in range(_HEADS):
                sl = slice(h * _HD, (h + 1) * _HD)
                qh = qb[:, sl]

                def inner(u, carry, qh=qh, sl=sl):
                    m, l, a = carry
                    kh = k_buf[pl.ds(u * _BLK, _BLK), sl]
                    vh = v_buf[pl.ds(u * _BLK, _BLK), sl]
                    s = _dot_t(qh, kh)
                    cg = (lo_a + u * _BLK
                          + jax.lax.broadcasted_iota(jnp.int32, (1, _BLK), 1))
                    cv = (cg >= lo) & (cg < hi)
                    sm = jnp.where(cv, s, _NEG)
                    m_new = jnp.maximum(m, jnp.max(sm, axis=1, keepdims=True))
                    p = jnp.where(cv, jnp.exp(sm - m_new), 0.0)
                    corr = jnp.exp(m - m_new)
                    a = a * corr + _dot(p, vh)
                    l = l * corr + jnp.sum(p, axis=1, keepdims=True)
                    return m_new, l, a

                m0 = jnp.full((_BLK, 1), _NEG, jnp.float32)
                l0 = jnp.zeros((_BLK, 1), jnp.float32)
                a0 = jnp.zeros((_BLK, _HD), jnp.float32)
                m, l, a = jax.lax.fori_loop(0, nblk, inner, (m0, l0, a0))
                m_f = jnp.maximum(m, 0.0)
                c1 = jnp.exp(m - m_f)
                c0 = jnp.exp(-m_f)
                num = a * c1 + c0 * vs_out[:, sl]
                den = l * c1 + c0 * n_out
                ctx = jnp.where(q_ins, num / den, 0.0)
                outs.append(jnp.sum(ctx, axis=0, keepdims=True))
            return acc + jnp.concatenate(outs, axis=1)

        acc_ref[...] = jax.lax.fori_loop(
            0, nblk, phase_b, jnp.zeros((1, _EMBED), jnp.float32))

    out_ref[...] = _dot_t(acc_ref[...] / nf, wo_ref[...]) + bo_ref[...]


def _call(pos_arr, ids2, x, wq, wk, wv, wo, bq2, bk2, bv2, bo2,
          interpret=False):
    return pl.pallas_call(
        _body,
        out_shape=jax.ShapeDtypeStruct((1, _EMBED), jnp.float32),
        in_specs=[
            pl.BlockSpec(memory_space=pltpu.SMEM),
            pl.BlockSpec(),
            pl.BlockSpec(),
            pl.BlockSpec(), pl.BlockSpec(), pl.BlockSpec(), pl.BlockSpec(),
            pl.BlockSpec(), pl.BlockSpec(), pl.BlockSpec(), pl.BlockSpec(),
        ],
        scratch_shapes=[
            pltpu.VMEM((_SEQ, _EMBED), jnp.float32),
            pltpu.VMEM((_SEQ, _EMBED), jnp.float32),
            pltpu.VMEM((_CAP, _EMBED), jnp.float32),
            pltpu.VMEM((_CAP, _HEADS * _CAP), jnp.float32),
            pltpu.VMEM((1, _EMBED), jnp.float32),
        ],
        interpret=interpret,
    )(pos_arr, ids2, x, wq, wk, wv, wo, bq2, bk2, bv2, bo2)


def kernel(x, segment_ids, pos, Wq, bq, Wk, bk, Wv, bv, Wo, bo):
    pos_arr = jnp.asarray(pos, jnp.int32).reshape(1, 1)
    ids2 = jnp.asarray(segment_ids, jnp.int32).reshape(_SEQ // 128, 128)
    out = _call(pos_arr, ids2, x,
                Wq, Wk, Wv, Wo,
                bq.reshape(1, _EMBED), bk.reshape(1, _EMBED),
                bv.reshape(1, _EMBED), bo.reshape(1, _EMBED))
    return out.reshape(_EMBED)


# speedup vs baseline: 18.3195x; 1.1658x over previous
import jax
import jax.numpy as jnp
from jax.experimental import pallas as pl
from jax.experimental.pallas import tpu as pltpu

_EMBED = 1024
_HEADS = 16
_HD = _EMBED // _HEADS
_SEQ = 2048
_BLK = 256
_CAP = 256
_NEG = -1e30


def _dot_t(a, b):
    return jax.lax.dot_general(
        a, b, (((1,), (1,)), ((), ())),
        preferred_element_type=jnp.float32,
        precision=jax.lax.Precision.DEFAULT)


def _dot(a, b):
    return jax.lax.dot_general(
        a, b, (((1,), (0,)), ((), ())),
        preferred_element_type=jnp.float32,
        precision=jax.lax.Precision.DEFAULT)


def _body(pos_ref, ids_ref, x_ref, wq_ref, wk_ref, wv_ref, wo_ref,
          bq_ref, bk_ref, bv_ref, bo_ref, out_ref,
          k_buf, v_buf, q_buf, s_buf, acc_ref):
    pos = pos_ref[0, 0]
    ids = ids_ref[...]
    ri = jax.lax.broadcasted_iota(jnp.int32, ids.shape, 0)
    ci = jax.lax.broadcasted_iota(jnp.int32, ids.shape, 1)
    flat = ri * 128 + ci
    seg = jnp.sum(jnp.where(flat == pos, ids, 0))
    lo = jnp.sum((ids < seg).astype(jnp.int32))
    n = jnp.sum((ids == seg).astype(jnp.int32))
    hi = lo + n
    nf = n.astype(jnp.float32)
    n_out = float(_SEQ) - nf

    bq = bq_ref[...]
    bk = bk_ref[...]
    bv = bv_ref[...]

    xsum = jnp.sum(x_ref[...], axis=0, keepdims=True)

    @pl.when(n <= _CAP - 7)
    def _fast():
        st = (jnp.minimum(lo, _SEQ - _CAP) // 8) * 8
        xw = x_ref[pl.ds(st, _CAP), :]
        g = st + jax.lax.broadcasted_iota(jnp.int32, (_CAP, 1), 0)
        ins = (g >= lo) & (g < hi)
        q_buf[...] = _dot_t(xw, wq_ref[...]) + bq
        k = _dot_t(xw, wk_ref[...]) + bk
        k_buf[pl.ds(0, _CAP), :] = jnp.where(ins, k, 0.0)
        xcat = jnp.concatenate([xw, xsum], axis=0)
        vfull = _dot_t(xcat, wv_ref[...]) + bv
        v = vfull[0:_CAP, :]
        v_buf[pl.ds(0, _CAP), :] = v
        vsum_all = vfull[_CAP:_CAP + 1, :] + float(_SEQ - 1) * bv
        v_win = jnp.sum(v, axis=0, keepdims=True)
        vs_out = vsum_all - v_win
        n_oow = float(_SEQ - _CAP)

        for h in range(_HEADS):
            qh = q_buf[:, h * _HD:(h + 1) * _HD]
            kh = k_buf[pl.ds(0, _CAP), h * _HD:(h + 1) * _HD]
            s_buf[:, h * _CAP:(h + 1) * _CAP] = _dot_t(qh, kh)

        s = s_buf[...]
        m_g = jnp.maximum(jnp.max(s), 0.0)
        em = jnp.exp(-m_g)
        s_buf[...] = jnp.exp(s - m_g)
        insf = ins.astype(jnp.float32)
        outs = []
        for h in range(_HEADS):
            ph = s_buf[:, h * _CAP:(h + 1) * _CAP]
            l = jnp.sum(ph, axis=1, keepdims=True)
            inv = insf / (l + em * n_oow)
            r = jnp.sum(ph * inv, axis=0, keepdims=True)
            vh = v_buf[pl.ds(0, _CAP), h * _HD:(h + 1) * _HD]
            alpha = jnp.sum(inv) * em
            outs.append(_dot(r, vh)
                        + alpha * vs_out[:, h * _HD:(h + 1) * _HD])
        acc_ref[...] = jnp.concatenate(outs, axis=1)

    @pl.when(n > _CAP - 7)
    def _general():
        vsum_all = _dot_t(xsum, wv_ref[...]) + float(_SEQ) * bv
        lo_a = (lo // _BLK) * _BLK
        nblk = (hi + _BLK - 1) // _BLK - lo // _BLK

        def phase_a(t, vs):
            base = pl.multiple_of(lo_a + t * _BLK, _BLK)
            xb = x_ref[pl.ds(base, _BLK), :]
            kb = _dot_t(xb, wk_ref[...]) + bk
            vb = _dot_t(xb, wv_ref[...]) + bv
            g = base + jax.lax.broadcasted_iota(jnp.int32, (_BLK, 1), 0)
            ins = (g >= lo) & (g < hi)
            kb = jnp.where(ins, kb, 0.0)
            vb = jnp.where(ins, vb, 0.0)
            k_buf[pl.ds(t * _BLK, _BLK), :] = kb
            v_buf[pl.ds(t * _BLK, _BLK), :] = vb
            return vs + jnp.sum(vb, axis=0, keepdims=True)

        vsum_seg = jax.lax.fori_loop(
            0, nblk, phase_a, jnp.zeros((1, _EMBED), jnp.float32))
        vs_out = vsum_all - vsum_seg

        def phase_b(t, acc):
            base = pl.multiple_of(lo_a + t * _BLK, _BLK)
            xb = x_ref[pl.ds(base, _BLK), :]
            qb = _dot_t(xb, wq_ref[...]) + bq
            g = base + jax.lax.broadcasted_iota(jnp.int32, (_BLK, 1), 0)
            q_ins = (g >= lo) & (g < hi)
            outs = []
            for h in range(_HEADS):
                sl = slice(h * _HD, (h + 1) * _HD)
                qh = qb[:, sl]

                def inner(u, carry, qh=qh, sl=sl):
                    m, l, a = carry
                    kh = k_buf[pl.ds(u * _BLK, _BLK), sl]
                    vh = v_buf[pl.ds(u * _BLK, _BLK), sl]
                    s = _dot_t(qh, kh)
                    cg = (lo_a + u * _BLK
                          + jax.lax.broadcasted_iota(jnp.int32, (1, _BLK), 1))
                    cv = (cg >= lo) & (cg < hi)
                    sm = jnp.where(cv, s, _NEG)
                    m_new = jnp.maximum(m, jnp.max(sm, axis=1, keepdims=True))
                    p = jnp.where(cv, jnp.exp(sm - m_new), 0.0)
                    corr = jnp.exp(m - m_new)
                    a = a * corr + _dot(p, vh)
                    l = l * corr + jnp.sum(p, axis=1, keepdims=True)
                    return m_new, l, a

                m0 = jnp.full((_BLK, 1), _NEG, jnp.float32)
                l0 = jnp.zeros((_BLK, 1), jnp.float32)
                a0 = jnp.zeros((_BLK, _HD), jnp.float32)
                m, l, a = jax.lax.fori_loop(0, nblk, inner, (m0, l0, a0))
                m_f = jnp.maximum(m, 0.0)
                c1 = jnp.exp(m - m_f)
                c0 = jnp.exp(-m_f)
                num = a * c1 + c0 * vs_out[:, sl]
                den = l * c1 + c0 * n_out
                ctx = jnp.where(q_ins, num / den, 0.0)
                outs.append(jnp.sum(ctx, axis=0, keepdims=True))
            return acc + jnp.concatenate(outs, axis=1)

        acc_ref[...] = jax.lax.fori_loop(
            0, nblk, phase_b, jnp.zeros((1, _EMBED), jnp.float32))

    out_ref[...] = _dot_t(acc_ref[...] / nf, wo_ref[...]) + bo_ref[...]


def _call(pos_arr, ids2, x, wq, wk, wv, wo, bq2, bk2, bv2, bo2,
          interpret=False):
    return pl.pallas_call(
        _body,
        out_shape=jax.ShapeDtypeStruct((1, _EMBED), jnp.float32),
        in_specs=[
            pl.BlockSpec(memory_space=pltpu.SMEM),
            pl.BlockSpec(),
            pl.BlockSpec(),
            pl.BlockSpec(), pl.BlockSpec(), pl.BlockSpec(), pl.BlockSpec(),
            pl.BlockSpec(), pl.BlockSpec(), pl.BlockSpec(), pl.BlockSpec(),
        ],
        scratch_shapes=[
            pltpu.VMEM((_SEQ, _EMBED), jnp.float32),
            pltpu.VMEM((_SEQ, _EMBED), jnp.float32),
            pltpu.VMEM((_CAP, _EMBED), jnp.float32),
            pltpu.VMEM((_CAP, _HEADS * _CAP), jnp.float32),
            pltpu.VMEM((1, _EMBED), jnp.float32),
        ],
        interpret=interpret,
    )(pos_arr, ids2, x, wq, wk, wv, wo, bq2, bk2, bv2, bo2)


def kernel(x, segment_ids, pos, Wq, bq, Wk, bk, Wv, bv, Wo, bo):
    pos_arr = jnp.asarray(pos, jnp.int32).reshape(1, 1)
    ids2 = jnp.asarray(segment_ids, jnp.int32).reshape(_SEQ // 128, 128)
    out = _call(pos_arr, ids2, x,
                Wq, Wk, Wv, Wo,
                bq.reshape(1, _EMBED), bk.reshape(1, _EMBED),
                bv.reshape(1, _EMBED), bo.reshape(1, _EMBED))
    return out.reshape(_EMBED)


# speedup vs baseline: 20.6429x; 1.1268x over previous
import jax
import jax.numpy as jnp
from jax.experimental import pallas as pl
from jax.experimental.pallas import tpu as pltpu

_EMBED = 1024
_HEADS = 16
_HD = _EMBED // _HEADS
_SEQ = 2048
_BLK = 256
_CAP = 256
_NEG = -1e30


def _dot_t(a, b):
    return jax.lax.dot_general(
        a, b, (((1,), (1,)), ((), ())),
        preferred_element_type=jnp.float32,
        precision=jax.lax.Precision.DEFAULT)


def _dot(a, b):
    return jax.lax.dot_general(
        a, b, (((1,), (0,)), ((), ())),
        preferred_element_type=jnp.float32,
        precision=jax.lax.Precision.DEFAULT)


def _body(pos_ref, ids_ref, bq_ref, bk_ref, bv_ref, bo_ref,
          x_hbm, wq_hbm, wk_hbm, wv_hbm, wo_hbm,
          out_ref,
          x_v, wq_v, wk_v, wv_v, wo_v,
          k_buf, v_buf, q_buf, s_buf, acc_ref,
          sem_x, sem_q, sem_k, sem_v, sem_o):
    cp_x = pltpu.make_async_copy(x_hbm, x_v, sem_x)
    cp_k = pltpu.make_async_copy(wk_hbm, wk_v, sem_k)
    cp_q = pltpu.make_async_copy(wq_hbm, wq_v, sem_q)
    cp_v = pltpu.make_async_copy(wv_hbm, wv_v, sem_v)
    cp_o = pltpu.make_async_copy(wo_hbm, wo_v, sem_o)
    cp_x.start()
    cp_k.start()
    cp_q.start()
    cp_v.start()
    cp_o.start()

    pos = pos_ref[0, 0]
    ids = ids_ref[...]
    ri = jax.lax.broadcasted_iota(jnp.int32, ids.shape, 0)
    ci = jax.lax.broadcasted_iota(jnp.int32, ids.shape, 1)
    flat = ri * 128 + ci
    seg = jnp.sum(jnp.where(flat == pos, ids, 0))
    lo = jnp.sum((ids < seg).astype(jnp.int32))
    n = jnp.sum((ids == seg).astype(jnp.int32))
    hi = lo + n
    nf = n.astype(jnp.float32)
    n_out = float(_SEQ) - nf

    bq = bq_ref[...]
    bk = bk_ref[...]
    bv = bv_ref[...]

    cp_x.wait()
    xsum = jnp.sum(x_v[...], axis=0, keepdims=True)

    @pl.when(n <= _CAP - 7)
    def _fast():
        st = (jnp.minimum(lo, _SEQ - _CAP) // 8) * 8
        xw = x_v[pl.ds(st, _CAP), :]
        g = st + jax.lax.broadcasted_iota(jnp.int32, (_CAP, 1), 0)
        ins = (g >= lo) & (g < hi)
        cp_k.wait()
        k = _dot_t(xw, wk_v[...]) + bk
        k_buf[pl.ds(0, _CAP), :] = jnp.where(ins, k, 0.0)
        cp_q.wait()
        q_buf[...] = _dot_t(xw, wq_v[...]) + bq

        for h in range(_HEADS):
            qh = q_buf[:, h * _HD:(h + 1) * _HD]
            kh = k_buf[pl.ds(0, _CAP), h * _HD:(h + 1) * _HD]
            s_buf[:, h * _CAP:(h + 1) * _CAP] = _dot_t(qh, kh)

        s = s_buf[...]
        m_g = jnp.maximum(jnp.max(s), 0.0)
        em = jnp.exp(-m_g)
        s_buf[...] = jnp.exp(s - m_g)
        insf = ins.astype(jnp.float32)
        n_oow = float(_SEQ - _CAP)
        rs = []
        alphas = []
        for h in range(_HEADS):
            ph = s_buf[:, h * _CAP:(h + 1) * _CAP]
            l = jnp.sum(ph, axis=1, keepdims=True)
            inv = insf / (l + em * n_oow)
            rs.append(jnp.sum(ph * inv, axis=0, keepdims=True))
            alphas.append(jnp.sum(inv) * em)

        cp_v.wait()
        xcat = jnp.concatenate([xw, xsum], axis=0)
        vfull = _dot_t(xcat, wv_v[...]) + bv
        v = vfull[0:_CAP, :]
        v_buf[pl.ds(0, _CAP), :] = v
        vsum_all = vfull[_CAP:_CAP + 1, :] + float(_SEQ - 1) * bv
        v_win = jnp.sum(v, axis=0, keepdims=True)
        vs_out = vsum_all - v_win
        outs = []
        for h in range(_HEADS):
            vh = v_buf[pl.ds(0, _CAP), h * _HD:(h + 1) * _HD]
            outs.append(_dot(rs[h], vh)
                        + alphas[h] * vs_out[:, h * _HD:(h + 1) * _HD])
        acc_ref[...] = jnp.concatenate(outs, axis=1)

    @pl.when(n > _CAP - 7)
    def _general():
        cp_k.wait()
        cp_v.wait()
        vsum_all = _dot_t(xsum, wv_v[...]) + float(_SEQ) * bv
        lo_a = (lo // _BLK) * _BLK
        nblk = (hi + _BLK - 1) // _BLK - lo // _BLK

        def phase_a(t, vs):
            base = pl.multiple_of(lo_a + t * _BLK, _BLK)
            xb = x_v[pl.ds(base, _BLK), :]
            kb = _dot_t(xb, wk_v[...]) + bk
            vb = _dot_t(xb, wv_v[...]) + bv
            g = base + jax.lax.broadcasted_iota(jnp.int32, (_BLK, 1), 0)
            ins = (g >= lo) & (g < hi)
            kb = jnp.where(ins, kb, 0.0)
            vb = jnp.where(ins, vb, 0.0)
            k_buf[pl.ds(t * _BLK, _BLK), :] = kb
            v_buf[pl.ds(t * _BLK, _BLK), :] = vb
            return vs + jnp.sum(vb, axis=0, keepdims=True)

        vsum_seg = jax.lax.fori_loop(
            0, nblk, phase_a, jnp.zeros((1, _EMBED), jnp.float32))
        vs_out = vsum_all - vsum_seg
        cp_q.wait()

        def phase_b(t, acc):
            base = pl.multiple_of(lo_a + t * _BLK, _BLK)
            xb = x_v[pl.ds(base, _BLK), :]
            qb = _dot_t(xb, wq_v[...]) + bq
            g = base + jax.lax.broadcasted_iota(jnp.int32, (_BLK, 1), 0)
            q_ins = (g >= lo) & (g < hi)
            outs = []
            for h in range(_HEADS):
                sl = slice(h * _HD, (h + 1) * _HD)
                qh = qb[:, sl]

                def inner(u, carry, qh=qh, sl=sl):
                    m, l, a = carry
                    kh = k_buf[pl.ds(u * _BLK, _BLK), sl]
                    vh = v_buf[pl.ds(u * _BLK, _BLK), sl]
                    s = _dot_t(qh, kh)
                    cg = (lo_a + u * _BLK
                          + jax.lax.broadcasted_iota(jnp.int32, (1, _BLK), 1))
                    cv = (cg >= lo) & (cg < hi)
                    sm = jnp.where(cv, s, _NEG)
                    m_new = jnp.maximum(m, jnp.max(sm, axis=1, keepdims=True))
                    p = jnp.where(cv, jnp.exp(sm - m_new), 0.0)
                    corr = jnp.exp(m - m_new)
                    a = a * corr + _dot(p, vh)
                    l = l * corr + jnp.sum(p, axis=1, keepdims=True)
                    return m_new, l, a

                m0 = jnp.full((_BLK, 1), _NEG, jnp.float32)
                l0 = jnp.zeros((_BLK, 1), jnp.float32)
                a0 = jnp.zeros((_BLK, _HD), jnp.float32)
                m, l, a = jax.lax.fori_loop(0, nblk, inner, (m0, l0, a0))
                m_f = jnp.maximum(m, 0.0)
                c1 = jnp.exp(m - m_f)
                c0 = jnp.exp(-m_f)
                num = a * c1 + c0 * vs_out[:, sl]
                den = l * c1 + c0 * n_out
                ctx = jnp.where(q_ins, num / den, 0.0)
                outs.append(jnp.sum(ctx, axis=0, keepdims=True))
            return acc + jnp.concatenate(outs, axis=1)

        acc_ref[...] = jax.lax.fori_loop(
            0, nblk, phase_b, jnp.zeros((1, _EMBED), jnp.float32))

    cp_o.wait()
    out_ref[...] = _dot_t(acc_ref[...] / nf, wo_v[...]) + bo_ref[...]


def _call(pos_arr, ids2, x, wq, wk, wv, wo, bq2, bk2, bv2, bo2,
          interpret=False):
    return pl.pallas_call(
        _body,
        out_shape=jax.ShapeDtypeStruct((1, _EMBED), jnp.float32),
        in_specs=[
            pl.BlockSpec(memory_space=pltpu.SMEM),
            pl.BlockSpec(),
            pl.BlockSpec(), pl.BlockSpec(),
            pl.BlockSpec(), pl.BlockSpec(),
            pl.BlockSpec(memory_space=pl.ANY),
            pl.BlockSpec(memory_space=pl.ANY),
            pl.BlockSpec(memory_space=pl.ANY),
            pl.BlockSpec(memory_space=pl.ANY),
            pl.BlockSpec(memory_space=pl.ANY),
        ],
        scratch_shapes=[
            pltpu.VMEM((_SEQ, _EMBED), jnp.float32),
            pltpu.VMEM((_EMBED, _EMBED), jnp.float32),
            pltpu.VMEM((_EMBED, _EMBED), jnp.float32),
            pltpu.VMEM((_EMBED, _EMBED), jnp.float32),
            pltpu.VMEM((_EMBED, _EMBED), jnp.float32),
            pltpu.VMEM((_SEQ, _EMBED), jnp.float32),
            pltpu.VMEM((_SEQ, _EMBED), jnp.float32),
            pltpu.VMEM((_CAP, _EMBED), jnp.float32),
            pltpu.VMEM((_CAP, _HEADS * _CAP), jnp.float32),
            pltpu.VMEM((1, _EMBED), jnp.float32),
            pltpu.SemaphoreType.DMA,
            pltpu.SemaphoreType.DMA,
            pltpu.SemaphoreType.DMA,
            pltpu.SemaphoreType.DMA,
            pltpu.SemaphoreType.DMA,
        ],
        interpret=interpret,
    )(pos_arr, ids2, bq2, bk2, bv2, bo2, x, wq, wk, wv, wo)


def kernel(x, segment_ids, pos, Wq, bq, Wk, bk, Wv, bv, Wo, bo):
    pos_arr = jnp.asarray(pos, jnp.int32).reshape(1, 1)
    ids2 = jnp.asarray(segment_ids, jnp.int32).reshape(_SEQ // 128, 128)
    out = _call(pos_arr, ids2, x,
                Wq, Wk, Wv, Wo,
                bq.reshape(1, _EMBED), bk.reshape(1, _EMBED),
                bv.reshape(1, _EMBED), bo.reshape(1, _EMBED))
    return out.reshape(_EMBED)
